# Initial kernel scaffold; baseline (speedup 1.0000x reference)
import jax
import jax.numpy as jnp
from jax.experimental import pallas as pl


def kernel(x, edge_index, edge_attr, batch, Wf, bf, Ws, bs, gbn, bbn, W1, b1, g2, b2, W2, bout):
    raise NotImplementedError("write your pallas kernel here")


# speedup vs baseline: 1.3681x; 1.3681x over previous
import dataclasses
import functools

import jax
import jax.numpy as jnp
from jax import lax
from jax.experimental import pallas as pl
from jax.experimental.pallas import tpu as pltpu
from jax.experimental.pallas import tpu_sc as plsc

N = 10000
E = 160000
F = 256
D = 16
G = 64

NC = 2
NS = 16
NW = NC * NS
HF = F // NC
CH = 100
GCH = 128
ROWS_PER_S = (E // CH) // NS
NPS = N // NS
NPAD = 10240

def _mesh():
    return plsc.VectorSubcoreMesh(core_axis_name="c", subcore_axis_name="s")


def _no_layout_cp():
    cp = pltpu.CompilerParams()
    if "needs_layout_passes" in pltpu.CompilerParams.__dataclass_fields__:
        cp = dataclasses.replace(cp, needs_layout_passes=False)
    return cp


def _bcast_lane(v, j):
    return lax.gather(
        v, jnp.full((16, 1), j, jnp.int32),
        lax.GatherDimensionNumbers(offset_dims=(), collapsed_slice_dims=(0,),
                                   start_index_map=(0,)),
        slice_sizes=(1,), mode=lax.GatherScatterMode.PROMISE_IN_BOUNDS)


def _mm_body(h_ref, wd_ref, ws_ref, pdf_ref, pds_ref, psf_ref, pss_ref):
    h = h_ref[...]
    pd = jnp.dot(h, wd_ref[...], preferred_element_type=jnp.float32)
    ps = jnp.dot(h, ws_ref[...], preferred_element_type=jnp.float32)
    pdf_ref[...] = pd[:, :F]
    pds_ref[...] = pd[:, F:]
    psf_ref[...] = ps[:, :F]
    pss_ref[...] = ps[:, F:]


def _node_matmul(h, Wd, Wsp):
    BM = 1000
    return pl.pallas_call(
        _mm_body,
        grid=(N // BM,),
        in_specs=[pl.BlockSpec((BM, F), lambda i: (i, 0)),
                  pl.BlockSpec((F, 2 * F), lambda i: (0, 0)),
                  pl.BlockSpec((F, 2 * F), lambda i: (0, 0))],
        out_specs=[pl.BlockSpec((BM, F), lambda i: (i, 0))] * 4,
        out_shape=[jax.ShapeDtypeStruct((N, F), jnp.float32)] * 4,
    )(h, Wd, Wsp)


def _sc_gather(Pdf, Pds, Psf, Pss, dst_g3, src_g3):
    NCK = E // GCH
    TRIPS = NCK // NS
    REM = NCK - TRIPS * NS

    @functools.partial(
        pl.kernel,
        out_type=[jax.ShapeDtypeStruct((NCK, GCH, F), jnp.float32)] * 4,
        mesh=_mesh(),
        scratch_types=[
            pltpu.VMEM((1, GCH), jnp.int32),
            pltpu.VMEM((1, GCH), jnp.int32),
            pltpu.VMEM((GCH, F), jnp.float32),
            pltpu.VMEM((GCH, F), jnp.float32),
            pltpu.SemaphoreType.DMA,
            pltpu.SemaphoreType.DMA,
        ],
    )
    def k(pdf_hbm, pds_hbm, psf_hbm, pss_hbm, d_hbm, s_hbm,
          gdf_hbm, gds_hbm, gsf_hbm, gss_hbm, di, si, bd, bs_, sem1, sem2):
        c = lax.axis_index("c")
        s = lax.axis_index("s")

        def do_chunk(ck, pd_hbm, ps_hbm, gd_hbm, gs_hbm):
            pltpu.sync_copy(d_hbm.at[ck], di)
            pltpu.sync_copy(s_hbm.at[ck], si)
            cd = pltpu.async_copy(pd_hbm.at[di.at[0]], bd, sem1)
            cs = pltpu.async_copy(ps_hbm.at[si.at[0]], bs_, sem2)
            cd.wait()
            cs.wait()
            pltpu.sync_copy(bd, gd_hbm.at[ck])
            pltpu.sync_copy(bs_, gs_hbm.at[ck])

        def do_all(pd_hbm, ps_hbm, gd_hbm, gs_hbm):
            @pl.loop(0, TRIPS)
            def _(t):
                do_chunk(s + NS * t, pd_hbm, ps_hbm, gd_hbm, gs_hbm)

            @pl.when(s < REM)
            def _():
                do_chunk(s + NS * TRIPS, pd_hbm, ps_hbm, gd_hbm, gs_hbm)

        @pl.when(c == 0)
        def _():
            do_all(pdf_hbm, psf_hbm, gdf_hbm, gsf_hbm)

        @pl.when(c == 1)
        def _():
            do_all(pds_hbm, pss_hbm, gds_hbm, gss_hbm)

    return k(Pdf, Pds, Psf, Pss, dst_g3, src_g3)


def _edge_body(gdf_ref, gds_ref, gsf_ref, gss_ref, ea_ref, we_ref,
               bf_ref, bs_ref, m_ref):
    ez = jnp.dot(ea_ref[...], we_ref[...], preferred_element_type=jnp.float32)
    zf = gdf_ref[...] + gsf_ref[...] + ez[:, :F] + bf_ref[...]
    zs = gds_ref[...] + gss_ref[...] + ez[:, F:] + bs_ref[...]
    sig = 1.0 / (1.0 + jnp.exp(-zf))
    sp = jnp.maximum(zs, 0.0) + jnp.log(1.0 + jnp.exp(-jnp.abs(zs)))
    m_ref[...] = sig * sp


def _edge_mlp(Gdf, Gds, Gsf, Gss, ea, We, bf_l, bs_l):
    BE = 1000
    return pl.pallas_call(
        _edge_body,
        grid=(E // BE,),
        in_specs=[pl.BlockSpec((BE, F), lambda i: (i, 0)),
                  pl.BlockSpec((BE, F), lambda i: (i, 0)),
                  pl.BlockSpec((BE, F), lambda i: (i, 0)),
                  pl.BlockSpec((BE, F), lambda i: (i, 0)),
                  pl.BlockSpec((BE, D), lambda i: (i, 0)),
                  pl.BlockSpec((D, 2 * F), lambda i: (0, 0)),
                  pl.BlockSpec((1, F), lambda i: (0, 0)),
                  pl.BlockSpec((1, F), lambda i: (0, 0))],
        out_specs=pl.BlockSpec((BE, F), lambda i: (i, 0)),
        out_shape=jax.ShapeDtypeStruct((E, F), jnp.float32),
    )(Gdf, Gds, Gsf, Gss, ea, We, bf_l, bs_l)


def _sc_scatter_add(m3, dst_s3):
    ZB = 32
    NPP = NPAD // NS

    @functools.partial(
        pl.kernel,
        out_type=jax.ShapeDtypeStruct((NPAD, F), jnp.float32),
        mesh=_mesh(),
        scratch_types=[
            pltpu.VMEM((ROWS_PER_S, CH), jnp.int32),
            pltpu.VMEM((CH, HF), jnp.float32),
            pltpu.VMEM((ZB, HF), jnp.float32),
            pltpu.VMEM_SHARED((NPAD, HF), jnp.float32),
        ],
    )
    def k(m_hbm, d_hbm, out_hbm, idx, buf, zbuf, acc):
        c = lax.axis_index("c")
        s = lax.axis_index("s")

        @pl.loop(0, ZB)
        def _(r):
            for kk in range(HF // 16):
                zbuf.at[r, pl.ds(kk * 16, 16)][...] = jnp.zeros(
                    (16,), jnp.float32)

        @pl.loop(0, NPP, step=ZB)
        def _(r):
            pltpu.sync_copy(zbuf, acc.at[pl.ds(s * NPP + r, ZB)])

        pltpu.sync_copy(d_hbm.at[s], idx)
        plsc.subcore_barrier()

        @pl.loop(0, ROWS_PER_S)
        def _(j):
            pltpu.sync_copy(m_hbm.at[s * ROWS_PER_S + j, :, pl.ds(c * HF, HF)],
                            buf)
            pltpu.sync_copy(buf, acc.at[idx.at[j]], add=True)

        plsc.subcore_barrier()
        pltpu.sync_copy(acc.at[pl.ds(s * NPP, NPP)],
                        out_hbm.at[pl.ds(s * NPP, NPP), pl.ds(c * HF, HF)])

    return k(m3, dst_s3)


def _bn_stats_body(a_ref, o_ref):
    i = pl.program_id(0)

    @pl.when(i == 0)
    def _():
        o_ref[...] = jnp.zeros_like(o_ref)

    a = a_ref[...]
    o_ref[0:1, :] += jnp.sum(a, axis=0, keepdims=True)
    o_ref[1:2, :] += jnp.sum(a * a, axis=0, keepdims=True)


def _bn_apply_body(a_ref, h_ref, st_ref, g_ref, b_ref, o_ref):
    mean = st_ref[0:1, :] * (1.0 / N)
    var = st_ref[1:2, :] * (1.0 / N) - mean * mean
    rstd = lax.rsqrt(var + 1e-5)
    o_ref[...] = (a_ref[...] - mean) * rstd * g_ref[...] + b_ref[...] + h_ref[...]


def _bn_residual(agg, h, g_l, b_l):
    BM = 1000
    stats = pl.pallas_call(
        _bn_stats_body,
        grid=(N // BM,),
        in_specs=[pl.BlockSpec((BM, F), lambda i: (i, 0))],
        out_specs=pl.BlockSpec((2, F), lambda i: (0, 0)),
        out_shape=jax.ShapeDtypeStruct((2, F), jnp.float32),
    )(agg)
    return pl.pallas_call(
        _bn_apply_body,
        grid=(N // BM,),
        in_specs=[pl.BlockSpec((BM, F), lambda i: (i, 0)),
                  pl.BlockSpec((BM, F), lambda i: (i, 0)),
                  pl.BlockSpec((2, F), lambda i: (0, 0)),
                  pl.BlockSpec((1, F), lambda i: (0, 0)),
                  pl.BlockSpec((1, F), lambda i: (0, 0))],
        out_specs=pl.BlockSpec((BM, F), lambda i: (i, 0)),
        out_shape=jax.ShapeDtypeStruct((N, F), jnp.float32),
    )(agg, h, stats, g_l, b_l)


def _sc_segment_max(h, batch):
    NCHUNK = N // 16
    TPS = (NCHUNK + NS - 1) // NS
    GPS = 8
    NRED = G // GPS

    @functools.partial(
        pl.kernel,
        out_type=jax.ShapeDtypeStruct((G, F), jnp.float32),
        mesh=_mesh(),
        compiler_params=_no_layout_cp(),
        scratch_types=[
            pltpu.VMEM((G, HF), jnp.float32),
            pltpu.VMEM((16, HF), jnp.float32),
            pltpu.VMEM((16,), jnp.int32),
            pltpu.VMEM((GPS, HF), jnp.float32),
            pltpu.VMEM((GPS, HF), jnp.float32),
            pltpu.VMEM_SHARED((NS * G, HF), jnp.float32),
        ],
    )
    def k(h_hbm, b_hbm, out_hbm, acc, rowbuf, idbuf, racc, tbuf, shacc):
        c = lax.axis_index("c")
        s = lax.axis_index("s")

        @pl.loop(0, G)
        def _(r):
            for kk in range(HF // 16):
                acc.at[r, pl.ds(kk * 16, 16)][...] = jnp.full(
                    (16,), -jnp.inf, jnp.float32)

        @pl.loop(0, TPS)
        def _(t):
            cid = s + NS * t

            @pl.when(cid < NCHUNK)
            def _():
                pltpu.sync_copy(b_hbm.at[pl.ds(cid * 16, 16)], idbuf)
                pltpu.sync_copy(
                    h_hbm.at[pl.ds(cid * 16, 16), pl.ds(c * HF, HF)], rowbuf)
                ids = idbuf[...]
                for j in range(16):
                    rsp = _bcast_lane(ids, j)
                    for kk in range(HF // 16):
                        colv = kk * 16 + lax.iota(jnp.int32, 16)
                        a = plsc.load_gather(acc, [rsp, colv])
                        r = rowbuf.at[j, pl.ds(kk * 16, 16)][...]
                        plsc.store_scatter(acc, [rsp, colv],
                                           jnp.maximum(a, r))

        pltpu.sync_copy(acc, shacc.at[pl.ds(s * G, G)])
        plsc.subcore_barrier()

        @pl.when(s < NRED)
        def _():
            pltpu.sync_copy(shacc.at[pl.ds(s * GPS, GPS)], racc)

            @pl.loop(1, NS)
            def _(t):
                pltpu.sync_copy(shacc.at[pl.ds(t * G + s * GPS, GPS)], tbuf)
                for rr in range(GPS):
                    for kk in range(HF // 16):
                        sl = (rr, pl.ds(kk * 16, 16))
                        racc.at[*sl][...] = jnp.maximum(racc.at[*sl][...],
                                                        tbuf.at[*sl][...])

            pltpu.sync_copy(
                racc, out_hbm.at[pl.ds(s * GPS, GPS), pl.ds(c * HF, HF)])

    return k(h, batch)


def _head_body(p_ref, w1_ref, b1_ref, g2_ref, b2_ref, w2_ref, bo_ref, o_ref):
    o1 = jnp.dot(p_ref[...], w1_ref[...], preferred_element_type=jnp.float32)
    o1 = jnp.maximum(o1 + b1_ref[...], 0.0)
    mean = jnp.mean(o1, axis=0, keepdims=True)
    var = jnp.mean((o1 - mean) ** 2, axis=0, keepdims=True)
    o1 = (o1 - mean) * lax.rsqrt(var + 1e-5) * g2_ref[...] + b2_ref[...]
    o_ref[...] = jnp.dot(o1, w2_ref[...],
                         preferred_element_type=jnp.float32) + bo_ref[...]


def _head(pooled, W1, b1, g2, b2, W2p, bout):
    DENSE = W1.shape[1]
    return pl.pallas_call(
        _head_body,
        out_shape=jax.ShapeDtypeStruct((G, 128), jnp.float32),
    )(pooled, W1, b1, g2, b2, W2p, bout)


def kernel(x, edge_index, edge_attr, batch, Wf, bf, Ws, bs, gbn, bbn,
           W1, b1, g2, b2, W2, bout):
    src = edge_index[0].astype(jnp.int32)
    dst = edge_index[1].astype(jnp.int32)
    dst_g3 = dst.reshape(E // GCH, 1, GCH)
    src_g3 = src.reshape(E // GCH, 1, GCH)
    dst_s3 = dst.reshape(NS, ROWS_PER_S, CH)

    h = x
    L = Wf.shape[0]
    for l in range(L):
        Wd = jnp.concatenate([Wf[l, 0:F], Ws[l, 0:F]], axis=1)
        Wsp = jnp.concatenate([Wf[l, F:2 * F], Ws[l, F:2 * F]], axis=1)
        We = jnp.concatenate([Wf[l, 2 * F:], Ws[l, 2 * F:]], axis=1)
        Pdf, Pds, Psf, Pss = _node_matmul(h, Wd, Wsp)
        Gdf, Gds, Gsf, Gss = _sc_gather(Pdf, Pds, Psf, Pss, dst_g3, src_g3)
        m = _edge_mlp(Gdf.reshape(E, F), Gds.reshape(E, F),
                      Gsf.reshape(E, F), Gss.reshape(E, F), edge_attr,
                      We, bf[l].reshape(1, F), bs[l].reshape(1, F))
        agg = _sc_scatter_add(m.reshape(E // CH, CH, F), dst_s3)
        h = _bn_residual(agg, h, gbn[l].reshape(1, F), bbn[l].reshape(1, F))

    pooled = _sc_segment_max(h, batch.astype(jnp.int32))
    out = _head(pooled, W1, b1.reshape(1, -1), g2.reshape(1, -1),
                b2.reshape(1, -1), jnp.pad(W2, ((0, 0), (0, 127))),
                bout.reshape(1, 1))
    return out[:, 0:1]
